# baseline (speedup 1.0000x reference)
import functools

import jax
import jax.numpy as jnp
from jax import lax
from jax.experimental import pallas as pl
from jax.experimental.pallas import tpu as pltpu
from jax.experimental.pallas import tpu_sc as plsc

_LODS = (16, 32, 64, 128)
_F = 16
_N = 1048576
_NC = 2
_NS = 16
_L = 16
_NW = _NC * _NS
_PER_W = _N // _NW
_CH = 128
_NCH = _PER_W // _CH
_NCHT = _N // _CH


def _sc_body(xyz_hbm, cb0, cb1, cb2, cb3, out_hbm,
             pbuf, ibuf, abuf, sbuf, psem, gsem, ssem):
    cid = lax.axis_index("c")
    sid = lax.axis_index("s")
    wid = sid * _NC + cid
    base_chunk = wid * _NCH
    cbs = (cb0, cb1, cb2, cb3)
    riota = lax.iota(jnp.int32, _L)
    zeros = jnp.zeros((_L,), jnp.float32)

    def pts_start(c, b):
        cn = jnp.minimum(c, _NCH - 1)
        pltpu.async_copy(xyz_hbm.at[base_chunk + cn], pbuf.at[b], psem.at[b])

    def pts_wait(b):
        pltpu.make_async_copy(
            xyz_hbm.at[base_chunk], pbuf.at[b], psem.at[b]).wait()

    def idx_compute(b):
        @pl.loop(0, _CH // _L)
        def _(i):
            s = pl.ds(i * _L, _L)
            x = pbuf[b, 0, s]
            y = pbuf[b, 1, s]
            z = pbuf[b, 2, s]
            for l, res in enumerate(_LODS):
                r = jnp.float32(res - 1)
                xi = (x * r).astype(jnp.int32)
                yi = (y * r).astype(jnp.int32)
                zi = (z * r).astype(jnp.int32)
                ibuf[b, l, s] = xi + yi * res + zi * (res * res)

    def gathers_start(b):
        for l in range(4):
            pltpu.async_copy(cbs[l].at[ibuf.at[b, l]], abuf.at[b],
                             gsem.at[b], add=True)

    def gathers_wait(b):
        for l in range(4):
            pltpu.make_async_copy(cbs[l].at[ibuf.at[b, l]], abuf.at[b],
                                  gsem.at[b]).wait()

    def zero_abuf(b):
        @pl.loop(0, _CH // _L)
        def _(g):
            ridx = riota + g * _L
            for f in range(_F):
                fidx = jnp.full((_L,), f, jnp.int32)
                plsc.store_scatter(abuf.at[b], [ridx, fidx], zeros)

    def tsum(b):
        @pl.loop(0, _CH // _L)
        def _(g):
            ridx = riota + g * _L
            for f in range(_F):
                fidx = jnp.full((_L,), f, jnp.int32)
                v = plsc.load_gather(abuf.at[b], [ridx, fidx])
                plsc.store_scatter(abuf.at[b], [ridx, fidx], zeros)
                sbuf[b, f // 8, f % 8, pl.ds(g * _L, _L)] = v

    def store_start(c, b):
        for r in range(2):
            pltpu.async_copy(sbuf.at[b, r], out_hbm.at[r, base_chunk + c],
                             ssem.at[b])

    def store_wait(b):
        for r in range(2):
            pltpu.make_async_copy(sbuf.at[b, r], out_hbm.at[r, base_chunk],
                                  ssem.at[b]).wait()

    zero_abuf(0)
    zero_abuf(1)
    pts_start(0, 0)
    pts_start(1, 1)
    pts_wait(0)
    idx_compute(0)
    pts_start(2, 0)
    gathers_start(0)
    pts_wait(1)
    idx_compute(1)
    pts_start(3, 1)
    gathers_start(1)
    gathers_wait(0)
    tsum(0)
    store_start(0, 0)
    pts_wait(0)
    idx_compute(0)
    pts_start(4, 0)
    gathers_start(0)
    gathers_wait(1)
    tsum(1)
    store_start(1, 1)
    pts_wait(1)
    idx_compute(1)
    pts_start(5, 1)
    gathers_start(1)
    gathers_wait(0)
    store_wait(0)
    tsum(0)
    store_start(2, 0)

    @pl.loop(0, (_NCH - 4) // 2)
    def _steady(cc):
        for b in range(2):
            c = 4 + 2 * cc + b
            pts_wait(b)
            idx_compute(b)
            pts_start(c + 2, b)
            gathers_start(b)
            gathers_wait(1 - b)
            store_wait(1 - b)
            tsum(1 - b)
            store_start(c - 1, 1 - b)

    gathers_wait(1)
    store_wait(1)
    tsum(1)
    store_start(_NCH - 1, 1)
    store_wait(0)
    store_wait(1)
    pts_wait(0)
    pts_wait(1)


@jax.jit
def _dense_grid_sc(xyz, cb0, cb1, cb2, cb3):
    mesh = plsc.VectorSubcoreMesh(core_axis_name="c", subcore_axis_name="s")
    return pl.kernel(
        _sc_body,
        out_type=jax.ShapeDtypeStruct((2, _NCHT, 8, _CH), jnp.float32),
        mesh=mesh,
        compiler_params=pltpu.CompilerParams(use_tc_tiling_on_sc=False,
                                             needs_layout_passes=False),
        scratch_types=[
            pltpu.VMEM((2, 3, _CH), jnp.float32),
            pltpu.VMEM((2, 4, _CH), jnp.int32),
            pltpu.VMEM((2, _CH, _F), jnp.float32),
            pltpu.VMEM((2, 2, 8, _CH), jnp.float32),
            pltpu.SemaphoreType.DMA((2,)),
            pltpu.SemaphoreType.DMA((2,)),
            pltpu.SemaphoreType.DMA((2,)),
        ],
    )(xyz, cb0, cb1, cb2, cb3)


def _rowmajor(cb):
    wide = jnp.concatenate([cb[k::8] for k in range(8)], axis=1)
    wide = lax.optimization_barrier(wide)
    return wide.reshape(-1, _F)


def kernel(pts, cb0, cb1, cb2, cb3):
    xyz = pts.T.reshape(3, _NCHT, _CH).transpose(1, 0, 2)
    cb0, cb1, cb2, cb3 = (_rowmajor(c) for c in (cb0, cb1, cb2, cb3))
    out4d = _dense_grid_sc(xyz, cb0, cb1, cb2, cb3)
    return out4d.transpose(1, 3, 0, 2).reshape(_N, _F)


# speedup vs baseline: 5.9096x; 5.9096x over previous
import functools

import jax
import jax.numpy as jnp
from jax import lax
from jax.experimental import pallas as pl
from jax.experimental.pallas import tpu as pltpu
from jax.experimental.pallas import tpu_sc as plsc

_LODS = (16, 32, 64, 128)
_F = 16
_N = 1048576
_NC = 2
_NS = 16
_L = 16
_NW = _NC * _NS
_PER_W = _N // _NW
_CH = 128
_NCH = _PER_W // _CH
_NCHT = _N // _CH
_VB = 512


def _sc_body(xyz_hbm, cbB0, cbB1, cbB2, cbB3,
             out_hbm, s0, s1, s2, s3,
             pbuf, ibuf, abuf, sbuf, tbuf, vbuf,
             psem, gsem, ssem, tsem, osem):
    cid = lax.axis_index("c")
    sid = lax.axis_index("s")
    wid = sid * _NC + cid
    base_chunk = wid * _NCH
    riota = lax.iota(jnp.int32, _L)
    ziota = jnp.zeros((_L,), jnp.int32)
    zeros = jnp.zeros((_L,), jnp.float32)
    fconst = [jnp.full((_L,), f, jnp.int32) for f in range(_F)]

    def trans_block(par, ngroups):
        @pl.loop(0, ngroups)
        def _(g):
            clsplat = ziota + (g >> 3)
            pciota = riota + ((g & 7) << 4)
            g16 = riota + g * _L
            for f in range(_F):
                v = plsc.load_gather(
                    tbuf.at[par, f // 8],
                    [clsplat, fconst[f % 8], pciota])
                plsc.store_scatter(vbuf.at[par], [g16, fconst[f]], v)

    def transpose_table(cbB, sref, nb):
        vl16 = nb * _VB
        cbase = sid * (vl16 // 128)
        vbase = sid * vl16

        def in_start(bk, par):
            c0 = cbase + bk * 4
            for r in range(2):
                pltpu.async_copy(cbB.at[r, pl.ds(c0, 4)],
                                 tbuf.at[par, r], tsem.at[par])

        def in_wait(par):
            for r in range(2):
                pltpu.make_async_copy(cbB.at[r, pl.ds(0, 4)],
                                      tbuf.at[par, r], tsem.at[par]).wait()

        def out_start(bk, par):
            pltpu.async_copy(vbuf.at[par],
                             sref.at[pl.ds(vbase + bk * _VB, _VB)],
                             osem.at[par])

        def out_wait(par):
            pltpu.make_async_copy(vbuf.at[par], sref.at[pl.ds(0, _VB)],
                                  osem.at[par]).wait()

        in_start(0, 0)
        in_start(1, 1)
        for par in range(2):
            in_wait(par)
            trans_block(par, _VB // _L)
            in_start(par + 2 if nb > 2 else nb - 1, par)
            out_start(par, par)

        if nb > 2:
            @pl.loop(0, (nb - 2) // 2)
            def _(bb):
                for par in range(2):
                    bk = 2 + 2 * bb + par
                    in_wait(par)
                    out_wait(par)
                    trans_block(par, _VB // _L)
                    nxt = jnp.minimum(bk + 2, nb - 1)
                    c0 = cbase + nxt * 4
                    for r in range(2):
                        pltpu.async_copy(cbB.at[r, pl.ds(c0, 4)],
                                         tbuf.at[par, r], tsem.at[par])
                    out_start(bk, par)

        out_wait(0)
        out_wait(1)
        in_wait(0)
        in_wait(1)

    def transpose_cb0():
        cbase = sid * 2
        for r in range(2):
            pltpu.sync_copy(cbB0.at[r, pl.ds(cbase, 2)],
                            tbuf.at[0, r, pl.ds(0, 2)])
        trans_block(0, 256 // _L)
        pltpu.sync_copy(vbuf.at[0, pl.ds(0, 256)],
                        s0.at[pl.ds(sid * 256, 256)])

    transpose_cb0()
    transpose_table(cbB1, s1, 32768 // _NS // _VB)
    transpose_table(cbB2, s2, 262144 // _NS // _VB)
    transpose_table(cbB3, s3, 2097152 // _NS // _VB)
    plsc.subcore_barrier()

    cbs = (s0, s1, s2, s3)

    def pts_start(c, b):
        cn = jnp.minimum(c, _NCH - 1)
        pltpu.async_copy(xyz_hbm.at[base_chunk + cn], pbuf.at[b], psem.at[b])

    def pts_wait(b):
        pltpu.make_async_copy(
            xyz_hbm.at[base_chunk], pbuf.at[b], psem.at[b]).wait()

    def idx_compute(b):
        @pl.loop(0, _CH // _L)
        def _(i):
            s = pl.ds(i * _L, _L)
            x = pbuf[b, 0, s]
            y = pbuf[b, 1, s]
            z = pbuf[b, 2, s]
            for l, res in enumerate(_LODS):
                r = jnp.float32(res - 1)
                xi = (x * r).astype(jnp.int32)
                yi = (y * r).astype(jnp.int32)
                zi = (z * r).astype(jnp.int32)
                ibuf[b, l, s] = xi + yi * res + zi * (res * res)

    def gathers_start(b):
        for l in range(4):
            pltpu.async_copy(cbs[l].at[ibuf.at[b, l]], abuf.at[b],
                             gsem.at[b], add=True)

    def gathers_wait(b):
        for l in range(4):
            pltpu.make_async_copy(cbs[l].at[ibuf.at[b, l]], abuf.at[b],
                                  gsem.at[b]).wait()

    def zero_abuf(b):
        @pl.loop(0, _CH // _L)
        def _(g):
            ridx = riota + g * _L
            for f in range(_F):
                plsc.store_scatter(abuf.at[b], [ridx, fconst[f]], zeros)

    def tsum(b):
        @pl.loop(0, _CH // _L)
        def _(g):
            ridx = riota + g * _L
            for f in range(_F):
                v = plsc.load_gather(abuf.at[b], [ridx, fconst[f]])
                plsc.store_scatter(abuf.at[b], [ridx, fconst[f]], zeros)
                sbuf[b, f // 8, f % 8, pl.ds(g * _L, _L)] = v

    def store_start(c, b):
        for r in range(2):
            pltpu.async_copy(sbuf.at[b, r], out_hbm.at[r, base_chunk + c],
                             ssem.at[b])

    def store_wait(b):
        for r in range(2):
            pltpu.make_async_copy(sbuf.at[b, r], out_hbm.at[r, base_chunk],
                                  ssem.at[b]).wait()

    zero_abuf(0)
    zero_abuf(1)
    pts_start(0, 0)
    pts_start(1, 1)
    pts_wait(0)
    idx_compute(0)
    pts_start(2, 0)
    gathers_start(0)
    pts_wait(1)
    idx_compute(1)
    pts_start(3, 1)
    gathers_start(1)
    gathers_wait(0)
    tsum(0)
    store_start(0, 0)
    pts_wait(0)
    idx_compute(0)
    pts_start(4, 0)
    gathers_start(0)
    gathers_wait(1)
    tsum(1)
    store_start(1, 1)
    pts_wait(1)
    idx_compute(1)
    pts_start(5, 1)
    gathers_start(1)
    gathers_wait(0)
    store_wait(0)
    tsum(0)
    store_start(2, 0)

    @pl.loop(0, (_NCH - 4) // 2)
    def _steady(cc):
        for b in range(2):
            c = 4 + 2 * cc + b
            pts_wait(b)
            idx_compute(b)
            pts_start(c + 2, b)
            gathers_start(b)
            gathers_wait(1 - b)
            store_wait(1 - b)
            tsum(1 - b)
            store_start(c - 1, 1 - b)

    gathers_wait(1)
    store_wait(1)
    tsum(1)
    store_start(_NCH - 1, 1)
    store_wait(0)
    store_wait(1)
    pts_wait(0)
    pts_wait(1)


@jax.jit
def _dense_grid_sc(xyz, cbB0, cbB1, cbB2, cbB3):
    mesh = plsc.VectorSubcoreMesh(core_axis_name="c", subcore_axis_name="s")
    return pl.kernel(
        _sc_body,
        out_type=[
            jax.ShapeDtypeStruct((2, _NCHT, 8, _CH), jnp.float32),
            jax.ShapeDtypeStruct((16 ** 3, _F), jnp.float32),
            jax.ShapeDtypeStruct((32 ** 3, _F), jnp.float32),
            jax.ShapeDtypeStruct((64 ** 3, _F), jnp.float32),
            jax.ShapeDtypeStruct((128 ** 3, _F), jnp.float32),
        ],
        mesh=mesh,
        compiler_params=pltpu.CompilerParams(use_tc_tiling_on_sc=False,
                                             needs_layout_passes=False),
        scratch_types=[
            pltpu.VMEM((2, 3, _CH), jnp.float32),
            pltpu.VMEM((2, 4, _CH), jnp.int32),
            pltpu.VMEM((2, _CH, _F), jnp.float32),
            pltpu.VMEM((2, 2, 8, _CH), jnp.float32),
            pltpu.VMEM((2, 2, 4, 8, 128), jnp.float32),
            pltpu.VMEM((2, _VB, _F), jnp.float32),
            pltpu.SemaphoreType.DMA((2,)),
            pltpu.SemaphoreType.DMA((2,)),
            pltpu.SemaphoreType.DMA((2,)),
            pltpu.SemaphoreType.DMA((2,)),
            pltpu.SemaphoreType.DMA((2,)),
        ],
    )(xyz, cbB0, cbB1, cbB2, cbB3)


def _tiled_view(cb):
    v = cb.shape[0]
    return cb.T.reshape(2, 8, v // 128, 128).transpose(0, 2, 1, 3)


def kernel(pts, cb0, cb1, cb2, cb3):
    xyz = pts.T.reshape(3, _NCHT, _CH).transpose(1, 0, 2)
    outs = _dense_grid_sc(xyz, _tiled_view(cb0), _tiled_view(cb1),
                          _tiled_view(cb2), _tiled_view(cb3))
    return outs[0].transpose(1, 3, 0, 2).reshape(_N, _F)


# speedup vs baseline: 5.9127x; 1.0005x over previous
import functools

import jax
import jax.numpy as jnp
from jax import lax
from jax.experimental import pallas as pl
from jax.experimental.pallas import tpu as pltpu
from jax.experimental.pallas import tpu_sc as plsc

_LODS = (16, 32, 64, 128)
_F = 16
_N = 1048576
_NC = 2
_NS = 16
_L = 16
_NW = _NC * _NS
_PER_W = _N // _NW
_CH = 128
_NCH = _PER_W // _CH
_NCHT = _N // _CH
_VB = 512


def _sc_body(xyz_hbm, cbB0, cbB1, cbB2, cbB3,
             out_hbm, s0, s1, s2, s3,
             pbuf, ibuf, abuf, sbuf, tbuf, vbuf,
             psem, gsem, ssem, tsem, osem):
    cid = lax.axis_index("c")
    sid = lax.axis_index("s")
    wid = sid * _NC + cid
    base_chunk = wid * _NCH
    riota = lax.iota(jnp.int32, _L)
    ziota = jnp.zeros((_L,), jnp.int32)
    zeros = jnp.zeros((_L,), jnp.float32)
    fconst = [jnp.full((_L,), f, jnp.int32) for f in range(_F)]

    def trans_block(par, ngroups):
        @pl.loop(0, ngroups, unroll=4)
        def _(g):
            clsplat = ziota + (g >> 3)
            pciota = riota + ((g & 7) << 4)
            g16 = riota + g * _L
            for f in range(_F):
                v = plsc.load_gather(
                    tbuf.at[par, f // 8],
                    [clsplat, fconst[f % 8], pciota])
                plsc.store_scatter(vbuf.at[par], [g16, fconst[f]], v)

    def transpose_table(cbB, sref, nb):
        vl16 = nb * _VB
        cbase = sid * (vl16 // 128)
        vbase = sid * vl16

        def in_start(bk, par):
            c0 = cbase + bk * 4
            for r in range(2):
                pltpu.async_copy(cbB.at[r, pl.ds(c0, 4)],
                                 tbuf.at[par, r], tsem.at[par])

        def in_wait(par):
            for r in range(2):
                pltpu.make_async_copy(cbB.at[r, pl.ds(0, 4)],
                                      tbuf.at[par, r], tsem.at[par]).wait()

        def out_start(bk, par):
            pltpu.async_copy(vbuf.at[par],
                             sref.at[pl.ds(vbase + bk * _VB, _VB)],
                             osem.at[par])

        def out_wait(par):
            pltpu.make_async_copy(vbuf.at[par], sref.at[pl.ds(0, _VB)],
                                  osem.at[par]).wait()

        in_start(0, 0)
        in_start(1, 1)
        for par in range(2):
            in_wait(par)
            trans_block(par, _VB // _L)
            in_start(par + 2 if nb > 2 else nb - 1, par)
            out_start(par, par)

        if nb > 2:
            @pl.loop(0, (nb - 2) // 2)
            def _(bb):
                for par in range(2):
                    bk = 2 + 2 * bb + par
                    in_wait(par)
                    out_wait(par)
                    trans_block(par, _VB // _L)
                    nxt = jnp.minimum(bk + 2, nb - 1)
                    c0 = cbase + nxt * 4
                    for r in range(2):
                        pltpu.async_copy(cbB.at[r, pl.ds(c0, 4)],
                                         tbuf.at[par, r], tsem.at[par])
                    out_start(bk, par)

        out_wait(0)
        out_wait(1)
        in_wait(0)
        in_wait(1)

    def transpose_cb0():
        cbase = sid * 2
        for r in range(2):
            pltpu.sync_copy(cbB0.at[r, pl.ds(cbase, 2)],
                            tbuf.at[0, r, pl.ds(0, 2)])
        trans_block(0, 256 // _L)
        pltpu.sync_copy(vbuf.at[0, pl.ds(0, 256)],
                        s0.at[pl.ds(sid * 256, 256)])

    transpose_cb0()
    transpose_table(cbB1, s1, 32768 // _NS // _VB)
    transpose_table(cbB2, s2, 262144 // _NS // _VB)
    transpose_table(cbB3, s3, 2097152 // _NS // _VB)
    plsc.subcore_barrier()

    cbs = (s0, s1, s2, s3)

    def pts_start(c, b):
        cn = jnp.minimum(c, _NCH - 1)
        pltpu.async_copy(xyz_hbm.at[base_chunk + cn], pbuf.at[b], psem.at[b])

    def pts_wait(b):
        pltpu.make_async_copy(
            xyz_hbm.at[base_chunk], pbuf.at[b], psem.at[b]).wait()

    def idx_compute(b):
        @pl.loop(0, _CH // _L, unroll=4)
        def _(i):
            s = pl.ds(i * _L, _L)
            x = pbuf[b, 0, s]
            y = pbuf[b, 1, s]
            z = pbuf[b, 2, s]
            for l, res in enumerate(_LODS):
                r = jnp.float32(res - 1)
                xi = (x * r).astype(jnp.int32)
                yi = (y * r).astype(jnp.int32)
                zi = (z * r).astype(jnp.int32)
                ibuf[b, l, s] = xi + yi * res + zi * (res * res)

    def gathers_start(b):
        for l in range(4):
            pltpu.async_copy(cbs[l].at[ibuf.at[b, l]], abuf.at[b],
                             gsem.at[b], add=True)

    def gathers_wait(b):
        for l in range(4):
            pltpu.make_async_copy(cbs[l].at[ibuf.at[b, l]], abuf.at[b],
                                  gsem.at[b]).wait()

    def zero_abuf(b):
        @pl.loop(0, _CH // _L, unroll=4)
        def _(g):
            ridx = riota + g * _L
            for f in range(_F):
                plsc.store_scatter(abuf.at[b], [ridx, fconst[f]], zeros)

    def tsum(b):
        @pl.loop(0, _CH // _L, unroll=4)
        def _(g):
            ridx = riota + g * _L
            for f in range(_F):
                v = plsc.load_gather(abuf.at[b], [ridx, fconst[f]])
                plsc.store_scatter(abuf.at[b], [ridx, fconst[f]], zeros)
                sbuf[b, f // 8, f % 8, pl.ds(g * _L, _L)] = v

    def store_start(c, b):
        for r in range(2):
            pltpu.async_copy(sbuf.at[b, r], out_hbm.at[r, base_chunk + c],
                             ssem.at[b])

    def store_wait(b):
        for r in range(2):
            pltpu.make_async_copy(sbuf.at[b, r], out_hbm.at[r, base_chunk],
                                  ssem.at[b]).wait()

    zero_abuf(0)
    zero_abuf(1)
    pts_start(0, 0)
    pts_start(1, 1)
    pts_wait(0)
    idx_compute(0)
    pts_start(2, 0)
    gathers_start(0)
    pts_wait(1)
    idx_compute(1)
    pts_start(3, 1)
    gathers_start(1)
    gathers_wait(0)
    tsum(0)
    store_start(0, 0)
    pts_wait(0)
    idx_compute(0)
    pts_start(4, 0)
    gathers_start(0)
    gathers_wait(1)
    tsum(1)
    store_start(1, 1)
    pts_wait(1)
    idx_compute(1)
    pts_start(5, 1)
    gathers_start(1)
    gathers_wait(0)
    store_wait(0)
    tsum(0)
    store_start(2, 0)

    @pl.loop(0, (_NCH - 4) // 2)
    def _steady(cc):
        for b in range(2):
            c = 4 + 2 * cc + b
            pts_wait(b)
            idx_compute(b)
            pts_start(c + 2, b)
            gathers_start(b)
            gathers_wait(1 - b)
            store_wait(1 - b)
            tsum(1 - b)
            store_start(c - 1, 1 - b)

    gathers_wait(1)
    store_wait(1)
    tsum(1)
    store_start(_NCH - 1, 1)
    store_wait(0)
    store_wait(1)
    pts_wait(0)
    pts_wait(1)


@jax.jit
def _dense_grid_sc(xyz, cbB0, cbB1, cbB2, cbB3):
    mesh = plsc.VectorSubcoreMesh(core_axis_name="c", subcore_axis_name="s")
    return pl.kernel(
        _sc_body,
        out_type=[
            jax.ShapeDtypeStruct((2, _NCHT, 8, _CH), jnp.float32),
            jax.ShapeDtypeStruct((16 ** 3, _F), jnp.float32),
            jax.ShapeDtypeStruct((32 ** 3, _F), jnp.float32),
            jax.ShapeDtypeStruct((64 ** 3, _F), jnp.float32),
            jax.ShapeDtypeStruct((128 ** 3, _F), jnp.float32),
        ],
        mesh=mesh,
        compiler_params=pltpu.CompilerParams(use_tc_tiling_on_sc=False,
                                             needs_layout_passes=False),
        scratch_types=[
            pltpu.VMEM((2, 3, _CH), jnp.float32),
            pltpu.VMEM((2, 4, _CH), jnp.int32),
            pltpu.VMEM((2, _CH, _F), jnp.float32),
            pltpu.VMEM((2, 2, 8, _CH), jnp.float32),
            pltpu.VMEM((2, 2, 4, 8, 128), jnp.float32),
            pltpu.VMEM((2, _VB, _F), jnp.float32),
            pltpu.SemaphoreType.DMA((2,)),
            pltpu.SemaphoreType.DMA((2,)),
            pltpu.SemaphoreType.DMA((2,)),
            pltpu.SemaphoreType.DMA((2,)),
            pltpu.SemaphoreType.DMA((2,)),
        ],
    )(xyz, cbB0, cbB1, cbB2, cbB3)


def _tiled_view(cb):
    v = cb.shape[0]
    return cb.T.reshape(2, 8, v // 128, 128).transpose(0, 2, 1, 3)


def kernel(pts, cb0, cb1, cb2, cb3):
    xyz = pts.T.reshape(3, _NCHT, _CH).transpose(1, 0, 2)
    outs = _dense_grid_sc(xyz, _tiled_view(cb0), _tiled_view(cb1),
                          _tiled_view(cb2), _tiled_view(cb3))
    return outs[0].transpose(1, 3, 0, 2).reshape(_N, _F)


# speedup vs baseline: 5.9234x; 1.0018x over previous
import functools

import jax
import jax.numpy as jnp
from jax import lax
from jax.experimental import pallas as pl
from jax.experimental.pallas import tpu as pltpu
from jax.experimental.pallas import tpu_sc as plsc

_LODS = (16, 32, 64, 128)
_F = 16
_N = 1048576
_NC = 2
_NS = 16
_L = 16
_NW = _NC * _NS
_PER_W = _N // _NW
_CH = 128
_NCH = _PER_W // _CH
_NCHT = _N // _CH
_VB = 512


def _sc_body(xyz_hbm, cbB0, cbB1, cbB2, cbB3,
             out_hbm, s0, s1, s2, s3,
             pbuf, ibuf, abuf, sbuf, tbuf, vbuf,
             psem, gsem, ssem, tsem, osem):
    cid = lax.axis_index("c")
    sid = lax.axis_index("s")
    wid = sid * _NC + cid
    base_chunk = wid * _NCH
    riota = lax.iota(jnp.int32, _L)
    ziota = jnp.zeros((_L,), jnp.int32)
    zeros = jnp.zeros((_L,), jnp.float32)
    fconst = [jnp.full((_L,), f, jnp.int32) for f in range(_F)]

    def trans_block(par, ngroups):
        @pl.loop(0, ngroups)
        def _(g):
            clsplat = ziota + (g >> 3)
            pciota = riota + ((g & 7) << 4)
            g16 = riota + g * _L
            for f in range(_F):
                v = plsc.load_gather(
                    tbuf.at[par, f // 8],
                    [clsplat, fconst[f % 8], pciota])
                plsc.store_scatter(vbuf.at[par], [g16, fconst[f]], v)

    def transpose_table(cbB, sref, nb):
        vl16 = nb * _VB
        cbase = sid * (vl16 // 128)
        vbase = sid * vl16

        def in_start(bk, par):
            c0 = cbase + bk * 4
            for r in range(2):
                pltpu.async_copy(cbB.at[r, pl.ds(c0, 4)],
                                 tbuf.at[par, r], tsem.at[par])

        def in_wait(par):
            for r in range(2):
                pltpu.make_async_copy(cbB.at[r, pl.ds(0, 4)],
                                      tbuf.at[par, r], tsem.at[par]).wait()

        def out_start(bk, par):
            pltpu.async_copy(vbuf.at[par],
                             sref.at[pl.ds(vbase + bk * _VB, _VB)],
                             osem.at[par])

        def out_wait(par):
            pltpu.make_async_copy(vbuf.at[par], sref.at[pl.ds(0, _VB)],
                                  osem.at[par]).wait()

        in_start(0, 0)
        in_start(1, 1)
        for par in range(2):
            in_wait(par)
            trans_block(par, _VB // _L)
            in_start(par + 2 if nb > 2 else nb - 1, par)
            out_start(par, par)

        if nb > 2:
            @pl.loop(0, (nb - 2) // 2)
            def _(bb):
                for par in range(2):
                    bk = 2 + 2 * bb + par
                    in_wait(par)
                    out_wait(par)
                    trans_block(par, _VB // _L)
                    nxt = jnp.minimum(bk + 2, nb - 1)
                    c0 = cbase + nxt * 4
                    for r in range(2):
                        pltpu.async_copy(cbB.at[r, pl.ds(c0, 4)],
                                         tbuf.at[par, r], tsem.at[par])
                    out_start(bk, par)

        out_wait(0)
        out_wait(1)
        in_wait(0)
        in_wait(1)

    def transpose_cb0():
        cbase = sid * 2
        for r in range(2):
            pltpu.sync_copy(cbB0.at[r, pl.ds(cbase, 2)],
                            tbuf.at[0, r, pl.ds(0, 2)])
        trans_block(0, 256 // _L)
        pltpu.sync_copy(vbuf.at[0, pl.ds(0, 256)],
                        s0.at[pl.ds(sid * 256, 256)])

    transpose_cb0()
    transpose_table(cbB1, s1, 32768 // _NS // _VB)
    transpose_table(cbB2, s2, 262144 // _NS // _VB)
    transpose_table(cbB3, s3, 2097152 // _NS // _VB)
    plsc.subcore_barrier()

    cbs = (s0, s1, s2, s3)

    def pts_start(c, b):
        cn = jnp.minimum(c, _NCH - 1)
        pltpu.async_copy(xyz_hbm.at[base_chunk + cn], pbuf.at[b], psem.at[b])

    def pts_wait(b):
        pltpu.make_async_copy(
            xyz_hbm.at[base_chunk], pbuf.at[b], psem.at[b]).wait()

    def idx_compute(b):
        @pl.loop(0, _CH // _L)
        def _(i):
            s = pl.ds(i * _L, _L)
            x = pbuf[b, 0, s]
            y = pbuf[b, 1, s]
            z = pbuf[b, 2, s]
            for l, res in enumerate(_LODS):
                r = jnp.float32(res - 1)
                xi = (x * r).astype(jnp.int32)
                yi = (y * r).astype(jnp.int32)
                zi = (z * r).astype(jnp.int32)
                ibuf[b, l, s] = xi + yi * res + zi * (res * res)

    def gathers_start(b):
        for l in range(4):
            pltpu.async_copy(cbs[l].at[ibuf.at[b, l]], abuf.at[b],
                             gsem.at[b], add=True)

    def gathers_wait(b):
        for l in range(4):
            pltpu.make_async_copy(cbs[l].at[ibuf.at[b, l]], abuf.at[b],
                                  gsem.at[b]).wait()

    def zero_abuf(b):
        @pl.loop(0, _CH // _L)
        def _(g):
            ridx = riota + g * _L
            for f in range(_F):
                plsc.store_scatter(abuf.at[b], [ridx, fconst[f]], zeros)

    def tsum(b):
        @pl.loop(0, _CH // _L)
        def _(g):
            ridx = riota + g * _L
            for f in range(_F):
                v = plsc.load_gather(abuf.at[b], [ridx, fconst[f]])
                plsc.store_scatter(abuf.at[b], [ridx, fconst[f]], zeros)
                sbuf[b, f // 8, f % 8, pl.ds(g * _L, _L)] = v

    def store_start(c, b):
        for r in range(2):
            pltpu.async_copy(sbuf.at[b, r], out_hbm.at[r, base_chunk + c],
                             ssem.at[b])

    def store_wait(b):
        for r in range(2):
            pltpu.make_async_copy(sbuf.at[b, r], out_hbm.at[r, base_chunk],
                                  ssem.at[b]).wait()

    zero_abuf(0)
    zero_abuf(1)
    pts_start(0, 0)
    pts_start(1, 1)
    pts_wait(0)
    idx_compute(0)
    pts_start(2, 0)
    gathers_start(0)
    pts_wait(1)
    idx_compute(1)
    pts_start(3, 1)
    gathers_start(1)
    gathers_wait(0)
    tsum(0)
    store_start(0, 0)
    pts_wait(0)
    idx_compute(0)
    pts_start(4, 0)
    gathers_start(0)
    gathers_wait(1)
    tsum(1)
    store_start(1, 1)
    pts_wait(1)
    idx_compute(1)
    pts_start(5, 1)
    gathers_start(1)
    gathers_wait(0)
    store_wait(0)
    tsum(0)
    store_start(2, 0)

    @pl.loop(0, (_NCH - 4) // 2)
    def _steady(cc):
        for b in range(2):
            c = 4 + 2 * cc + b
            pts_wait(b)
            idx_compute(b)
            pts_start(c + 2, b)
            gathers_start(b)
            gathers_wait(1 - b)
            store_wait(1 - b)
            tsum(1 - b)
            store_start(c - 1, 1 - b)

    gathers_wait(1)
    store_wait(1)
    tsum(1)
    store_start(_NCH - 1, 1)
    store_wait(0)
    store_wait(1)
    pts_wait(0)
    pts_wait(1)


@jax.jit
def _dense_grid_sc(xyz, cbB0, cbB1, cbB2, cbB3):
    mesh = plsc.VectorSubcoreMesh(core_axis_name="c", subcore_axis_name="s")
    return pl.kernel(
        _sc_body,
        out_type=[
            jax.ShapeDtypeStruct((2, _NCHT, 8, _CH), jnp.float32),
            jax.ShapeDtypeStruct((16 ** 3, _F), jnp.float32),
            jax.ShapeDtypeStruct((32 ** 3, _F), jnp.float32),
            jax.ShapeDtypeStruct((64 ** 3, _F), jnp.float32),
            jax.ShapeDtypeStruct((128 ** 3, _F), jnp.float32),
        ],
        mesh=mesh,
        compiler_params=pltpu.CompilerParams(use_tc_tiling_on_sc=False,
                                             needs_layout_passes=False),
        scratch_types=[
            pltpu.VMEM((2, 3, _CH), jnp.float32),
            pltpu.VMEM((2, 4, _CH), jnp.int32),
            pltpu.VMEM((2, _CH, _F), jnp.float32),
            pltpu.VMEM((2, 2, 8, _CH), jnp.float32),
            pltpu.VMEM((2, 2, 4, 8, 128), jnp.float32),
            pltpu.VMEM((2, _VB, _F), jnp.float32),
            pltpu.SemaphoreType.DMA((2,)),
            pltpu.SemaphoreType.DMA((2,)),
            pltpu.SemaphoreType.DMA((2,)),
            pltpu.SemaphoreType.DMA((2,)),
            pltpu.SemaphoreType.DMA((2,)),
        ],
    )(xyz, cbB0, cbB1, cbB2, cbB3)


def _tiled_view(cb):
    v = cb.shape[0]
    return cb.T.reshape(2, 8, v // 128, 128).transpose(0, 2, 1, 3)


def kernel(pts, cb0, cb1, cb2, cb3):
    xyz = pts.T.reshape(3, _NCHT, _CH).transpose(1, 0, 2)
    outs = _dense_grid_sc(xyz, _tiled_view(cb0), _tiled_view(cb1),
                          _tiled_view(cb2), _tiled_view(cb3))
    return outs[0].transpose(1, 3, 0, 2).reshape(_N, _F)
